# baseline (speedup 1.0000x reference)
import functools

import jax
import jax.numpy as jnp
from jax import lax
from jax.experimental import pallas as pl
from jax.experimental.pallas import tpu as pltpu
from jax.experimental.pallas import tpu_sc as plsc

RH = 32


def _conv_body(up_ref, cen_ref, dn_ref, w_ref, b_ref, y_ref, z_scr):
    rb = pl.program_id(1)
    nrb = pl.num_programs(1)
    W = y_ref.shape[-1]
    for lu in range(RH + 2):
        if lu == 0:
            xr = up_ref[0, :, 7, :]
            xr = jnp.where(rb == 0, 0.0, xr)
        elif lu == RH + 1:
            xr = dn_ref[0, :, 0, :]
            xr = jnp.where(rb == nrb - 1, 0.0, xr)
        else:
            xr = cen_ref[0, :, lu - 1, :]
        z_scr[:, lu, :] = jnp.dot(w_ref[...], xr, preferred_element_type=jnp.float32)
    acc = jnp.zeros((8, RH, W), jnp.float32)
    for dy in range(3):
        for dx in range(3):
            k = dy * 3 + dx
            z = z_scr[k * 8:(k + 1) * 8, dy:dy + RH, :]
            sh = 1 - dx
            if sh == 1:
                z = jnp.concatenate([jnp.zeros((8, RH, 1), jnp.float32), z[:, :, :-1]], axis=-1)
            elif sh == -1:
                z = jnp.concatenate([z[:, :, 1:], jnp.zeros((8, RH, 1), jnp.float32)], axis=-1)
            acc = acc + z
    acc = acc + b_ref[...].reshape(8, 1, W)
    y_ref[0] = jnp.maximum(acc, 0.0)


def _conv_pallas(x, Wconv, bconv):
    B, C, H, W = x.shape
    nrb = H // RH
    Wall = Wconv.transpose(2, 3, 0, 1).reshape(9 * 8, C)
    return pl.pallas_call(
        _conv_body,
        grid=(B, nrb),
        in_specs=[
            pl.BlockSpec((1, C, 8, W), lambda b, rb: (b, 0, jnp.maximum(4 * rb - 1, 0), 0)),
            pl.BlockSpec((1, C, RH, W), lambda b, rb: (b, 0, rb, 0)),
            pl.BlockSpec((1, C, 8, W), lambda b, rb: (b, 0, jnp.minimum(4 * rb + 4, 27), 0)),
            pl.BlockSpec((9 * 8, C), lambda b, rb: (0, 0)),
            pl.BlockSpec((8, W), lambda b, rb: (0, 0)),
        ],
        out_specs=pl.BlockSpec((1, 8, RH, W), lambda b, rb: (b, 0, rb, 0)),
        out_shape=jax.ShapeDtypeStruct((B, 8, H, W), jnp.float32),
        scratch_shapes=[pltpu.VMEM((9 * 8, RH + 2, W), jnp.float32)],
    )(x, x, x, Wall, jnp.broadcast_to(bconv[:, None], (8, W)))


_NP = 196


def _cluster_body(f_hbm, q_hbm, spf_hbm,
                  ftile, imgbuf, qtile, snbr, spf3, part, tmp, kacc, rball, spfrow,
                  shp, sspf):
    c = lax.axis_index("c")
    s = lax.axis_index("s")
    iota = lax.iota(jnp.int32, 16)
    idx_m1 = jnp.maximum(iota - 1, 0)
    idx_p1 = jnp.minimum(iota + 1, 15)
    zero = jnp.zeros((16,), jnp.float32)
    sm1 = jnp.maximum(s - 1, 0)
    sp1 = jnp.minimum(s + 1, 15)

    iota14 = iota * 14

    for b in range(2):
        bg = 2 * c + b
        for cc in range(8):
            pltpu.sync_copy(f_hbm.at[bg, cc, s],
                            imgbuf.at[pl.ds(cc * 3200, 3200)])
        def rlbody(i, _c, b=b):
            cc = i // _NP
            p = i - cc * _NP
            py = p // 14
            px = p - py * 14
            v = plsc.load_gather(
                imgbuf, [iota14 + (cc * 3200 + py * 224 + px)])
            ftile[b, pl.ds(p * 128 + cc * 16, 16)] = v
            return _c
        lax.fori_loop(0, 8 * _NP, rlbody, 0)

    for b in range(2):
        def ibody(p, accs):
            return tuple(a + ftile[b, pl.ds(p * 128 + cc * 16, 16)]
                         for cc, a in enumerate(accs))
        accs = lax.fori_loop(0, _NP, ibody, (zero,) * 8)
        for cc in range(8):
            spfrow[pl.ds((b * 8 + cc) * 16, 16)] = accs[cc] * (1.0 / _NP)
        pltpu.sync_copy(spfrow.at[pl.ds(b * 128, 128)],
                        sspf.at[pl.ds((b * 16 + s) * 128, 128)])
    plsc.subcore_barrier()

    for t in range(5):
        for b in range(2):
            bg = 2 * c + b
            pltpu.sync_copy(sspf.at[pl.ds((b * 16 + sm1) * 128, 128)],
                            spf3.at[pl.ds(0, 128)])
            pltpu.sync_copy(sspf.at[pl.ds((b * 16 + s) * 128, 128)],
                            spf3.at[pl.ds(128, 128)])
            pltpu.sync_copy(sspf.at[pl.ds((b * 16 + sp1) * 128, 128)],
                            spf3.at[pl.ds(256, 128)])

            def gbody(i, _c):
                k = i // 8
                cc = i - k * 8
                dy = k // 3
                dx = k - dy * 3
                base = jnp.full((16,), 16, jnp.int32) * (dy * 8 + cc)
                idxv = jnp.where(dx == 0, idx_m1,
                                 jnp.where(dx == 1, iota, idx_p1))
                snbr[pl.ds(i * 16, 16)] = plsc.load_gather(spf3, [base + idxv])
                return _c
            lax.fori_loop(0, 72, gbody, 0)

            for i in range(81):
                kacc[pl.ds(i * 16, 16)] = zero

            def p1(p, _c, b=b):
                fs = [ftile[b, pl.ds(p * 128 + cc * 16, 16)] for cc in range(8)]
                ssqs = []
                for k in range(9):
                    ssq = zero
                    for cc in range(8):
                        d = fs[cc] - snbr[pl.ds((k * 8 + cc) * 16, 16)]
                        ssq = ssq + d * d
                    ssqs.append(ssq)
                m = ssqs[0]
                for k in range(1, 9):
                    m = jnp.minimum(m, ssqs[k])
                es = [jnp.exp(m - v) for v in ssqs]
                tot = es[0]
                for k in range(1, 9):
                    tot = tot + es[k]
                rinv = 1.0 / tot
                for k in range(9):
                    q = es[k] * rinv
                    qtile[pl.ds(k * (_NP * 16) + p * 16, 16)] = q
                    for cc in range(8):
                        plsc.addupdate(kacc.at[pl.ds((k * 9 + cc) * 16, 16)],
                                       q * fs[cc])
                    plsc.addupdate(kacc.at[pl.ds((k * 9 + 8) * 16, 16)], q)
                return _c
            lax.fori_loop(0, _NP, p1, 0)

            if t == 4:
                for k in range(9):
                    def qlbody(p, _c, k=k):
                        v = qtile[pl.ds(k * 3136 + p * 16, 16)]
                        py = p // 14
                        px = p - py * 14
                        plsc.store_scatter(
                            imgbuf, [iota14 + (py * 224 + px)], v)
                        return _c
                    lax.fori_loop(0, _NP, qlbody, 0)
                    pltpu.sync_copy(imgbuf.at[pl.ds(0, 3200)],
                                    q_hbm.at[bg, k, s])

            for i in range(27):
                part[pl.ds(i * 16, 16)] = zero

            def p2k(k, _c):
                dy = k // 3
                dx = k - dy * 3
                for ch in range(9):
                    tmp[ch, :] = kacc[pl.ds((k * 9 + ch) * 16, 16)]
                for ch in range(9):
                    chf = jnp.full((16,), ch, jnp.int32)
                    v = tmp[ch, :]
                    gA = plsc.load_gather(tmp, [chf, idx_p1])
                    shA = (jnp.where(iota <= 14, gA, 0.0)
                           + jnp.where(iota == 0, v, 0.0))
                    gB = plsc.load_gather(tmp, [chf, idx_m1])
                    shB = (jnp.where(iota >= 1, gB, 0.0)
                           + jnp.where(iota == 15, v, 0.0))
                    sh = jnp.where(dx == 0, shA, jnp.where(dx == 2, shB, v))
                    off = (dy * 9 + ch) * 16
                    part[pl.ds(off, 16)] = part[pl.ds(off, 16)] + sh
                return _c
            lax.fori_loop(0, 9, p2k, 0)

            is0 = s == 0
            is15 = s == 15
            for ch in range(9):
                p0 = part[pl.ds(ch * 16, 16)]
                p1v = part[pl.ds((9 + ch) * 16, 16)]
                p2v = part[pl.ds((18 + ch) * 16, 16)]
                p1n = (p1v + jnp.where(is0, p0, zero)
                       + jnp.where(is15, p2v, zero))
                part[pl.ds(ch * 16, 16)] = jnp.where(is0, zero, p0)
                part[pl.ds((9 + ch) * 16, 16)] = p1n
                part[pl.ds((18 + ch) * 16, 16)] = jnp.where(is15, zero, p2v)
            pltpu.sync_copy(part, shp.at[pl.ds((s * 2 + b) * 432, 432)])
        plsc.subcore_barrier()

        for b in range(2):
            bg = 2 * c + b
            pltpu.sync_copy(shp.at[pl.ds((s * 2 + b) * 432 + 144, 144)],
                            rball.at[pl.ds(0, 144)])
            pltpu.sync_copy(shp.at[pl.ds((sm1 * 2 + b) * 432 + 288, 144)],
                            rball.at[pl.ds(144, 144)])
            pltpu.sync_copy(shp.at[pl.ds((sp1 * 2 + b) * 432, 144)],
                            rball.at[pl.ds(288, 144)])
            up_ok = s > 0
            dn_ok = s < 15

            def rd(ch):
                a0 = rball[pl.ds(ch * 16, 16)]
                a1 = jnp.where(up_ok, rball[pl.ds((9 + ch) * 16, 16)], zero)
                a2 = jnp.where(dn_ok, rball[pl.ds((18 + ch) * 16, 16)], zero)
                return a0 + a1 + a2
            rinv = 1.0 / (rd(8) + 1e-8)
            for cc in range(8):
                spfrow[pl.ds((b * 8 + cc) * 16, 16)] = rd(cc) * rinv
            pltpu.sync_copy(spfrow.at[pl.ds(b * 128, 128)],
                            sspf.at[pl.ds((b * 16 + s) * 128, 128)])
            if t == 4:
                pltpu.sync_copy(spfrow.at[pl.ds(b * 128, 128)], spf_hbm.at[bg, s])
        plsc.subcore_barrier()


def _cluster_sc(fcell, interpret=False):
    mesh = plsc.VectorSubcoreMesh(core_axis_name="c", subcore_axis_name="s")
    f = pl.kernel(
        _cluster_body,
        out_type=[jax.ShapeDtypeStruct((4, 9, 16, 3200), jnp.float32),
                  jax.ShapeDtypeStruct((4, 16, 128), jnp.float32)],
        mesh=mesh,
        scratch_types=[
            pltpu.VMEM((2, _NP * 128), jnp.float32),
            pltpu.VMEM((25600,), jnp.float32),
            pltpu.VMEM((9 * _NP * 16,), jnp.float32),
            pltpu.VMEM((72 * 16,), jnp.float32),
            pltpu.VMEM((24 * 16,), jnp.float32),
            pltpu.VMEM((27 * 16,), jnp.float32),
            pltpu.VMEM((9, 16), jnp.float32),
            pltpu.VMEM((81 * 16,), jnp.float32),
            pltpu.VMEM((27 * 16,), jnp.float32),
            pltpu.VMEM((16 * 16,), jnp.float32),
            pltpu.VMEM_SHARED((32 * 27 * 16,), jnp.float32),
            pltpu.VMEM_SHARED((2 * 16 * 128,), jnp.float32),
        ],
        compiler_params=pltpu.CompilerParams(needs_layout_passes=False),
        interpret=interpret,
    )
    return f(fcell)


def kernel(featlist, Wconv, bconv):
    x = featlist[0]
    featall = _conv_pallas(x, Wconv, bconv)
    B, C, H, W = featall.shape
    f_pad = jnp.pad(featall.reshape(B, C, 16, 3136),
                    ((0, 0), (0, 0), (0, 0), (0, 64)))
    q_img, spf_cells = _cluster_sc(f_pad)
    q_img = q_img[:, :, :, :3136]
    Qout = q_img.reshape(B, 9, H, W)
    spf_out = (spf_cells.reshape(B, 16, 8, 16)
               .transpose(0, 1, 3, 2).reshape(B, 256, C))
    return (Qout, spf_out, featall)


# speedup vs baseline: 1.0911x; 1.0911x over previous
import functools

import jax
import jax.numpy as jnp
from jax import lax
from jax.experimental import pallas as pl
from jax.experimental.pallas import tpu as pltpu
from jax.experimental.pallas import tpu_sc as plsc

RH = 56


def _conv_body(up_ref, cen_ref, dn_ref, w_ref, b_ref, y_ref, z_scr):
    rb = pl.program_id(1)
    nrb = pl.num_programs(1)
    W = y_ref.shape[-1]
    for lu in range(RH + 2):
        if lu == 0:
            xr = up_ref[0, :, 7, :]
            xr = jnp.where(rb == 0, 0.0, xr)
        elif lu == RH + 1:
            xr = dn_ref[0, :, 0, :]
            xr = jnp.where(rb == nrb - 1, 0.0, xr)
        else:
            xr = cen_ref[0, :, lu - 1, :]
        z_scr[:, lu, :] = jnp.dot(w_ref[...], xr, preferred_element_type=jnp.float32)
    acc = jnp.zeros((8, RH, W), jnp.float32)
    for dy in range(3):
        for dx in range(3):
            k = dy * 3 + dx
            z = z_scr[k * 8:(k + 1) * 8, dy:dy + RH, :]
            sh = 1 - dx
            if sh == 1:
                z = jnp.concatenate([jnp.zeros((8, RH, 1), jnp.float32), z[:, :, :-1]], axis=-1)
            elif sh == -1:
                z = jnp.concatenate([z[:, :, 1:], jnp.zeros((8, RH, 1), jnp.float32)], axis=-1)
            acc = acc + z
    acc = acc + b_ref[...].reshape(8, 1, W)
    y_ref[0] = jnp.maximum(acc, 0.0)


def _conv_pallas(x, Wconv, bconv):
    B, C, H, W = x.shape
    nrb = H // RH
    Wall = Wconv.transpose(2, 3, 0, 1).reshape(9 * 8, C)
    return pl.pallas_call(
        _conv_body,
        grid=(B, nrb),
        in_specs=[
            pl.BlockSpec((1, C, 8, W), lambda b, rb: (b, 0, jnp.maximum(7 * rb - 1, 0), 0)),
            pl.BlockSpec((1, C, RH, W), lambda b, rb: (b, 0, rb, 0)),
            pl.BlockSpec((1, C, 8, W), lambda b, rb: (b, 0, jnp.minimum(7 * rb + 7, 27), 0)),
            pl.BlockSpec((9 * 8, C), lambda b, rb: (0, 0)),
            pl.BlockSpec((8, W), lambda b, rb: (0, 0)),
        ],
        out_specs=pl.BlockSpec((1, 8, RH, W), lambda b, rb: (b, 0, rb, 0)),
        out_shape=jax.ShapeDtypeStruct((B, 8, H, W), jnp.float32),
        scratch_shapes=[pltpu.VMEM((9 * 8, RH + 2, W), jnp.float32)],
    )(x, x, x, Wall, jnp.broadcast_to(bconv[:, None], (8, W)))


_NP = 196


def _cluster_body(f_hbm, q_hbm, spf_hbm,
                  ftile, imgbuf, qtile, snbr, spf3, part, tmp, kacc, rball, spfrow,
                  shp, sspf):
    c = lax.axis_index("c")
    s = lax.axis_index("s")
    iota = lax.iota(jnp.int32, 16)
    idx_m1 = jnp.maximum(iota - 1, 0)
    idx_p1 = jnp.minimum(iota + 1, 15)
    zero = jnp.zeros((16,), jnp.float32)
    sm1 = jnp.maximum(s - 1, 0)
    sp1 = jnp.minimum(s + 1, 15)

    iota14 = iota * 14

    for b in range(2):
        bg = 2 * c + b
        for cc in range(8):
            pltpu.sync_copy(f_hbm.at[bg, cc, s],
                            imgbuf.at[pl.ds(cc * 3200, 3200)])
        def rlbody(i, _c, b=b):
            cc = i // _NP
            p = i - cc * _NP
            py = p // 14
            px = p - py * 14
            v = plsc.load_gather(
                imgbuf, [iota14 + (cc * 3200 + py * 224 + px)])
            ftile[b, pl.ds(p * 128 + cc * 16, 16)] = v
            return _c
        lax.fori_loop(0, 8 * _NP, rlbody, 0)

    for b in range(2):
        def ibody(p, accs):
            return tuple(a + ftile[b, pl.ds(p * 128 + cc * 16, 16)]
                         for cc, a in enumerate(accs))
        accs = lax.fori_loop(0, _NP, ibody, (zero,) * 8)
        for cc in range(8):
            spfrow[pl.ds((b * 8 + cc) * 16, 16)] = accs[cc] * (1.0 / _NP)
        pltpu.sync_copy(spfrow.at[pl.ds(b * 128, 128)],
                        sspf.at[pl.ds((b * 16 + s) * 128, 128)])
    plsc.subcore_barrier()

    for t in range(5):
        for b in range(2):
            bg = 2 * c + b
            pltpu.sync_copy(sspf.at[pl.ds((b * 16 + sm1) * 128, 128)],
                            spf3.at[pl.ds(0, 128)])
            pltpu.sync_copy(sspf.at[pl.ds((b * 16 + s) * 128, 128)],
                            spf3.at[pl.ds(128, 128)])
            pltpu.sync_copy(sspf.at[pl.ds((b * 16 + sp1) * 128, 128)],
                            spf3.at[pl.ds(256, 128)])

            def gbody(i, _c):
                k = i // 8
                cc = i - k * 8
                dy = k // 3
                dx = k - dy * 3
                base = jnp.full((16,), 16, jnp.int32) * (dy * 8 + cc)
                idxv = jnp.where(dx == 0, idx_m1,
                                 jnp.where(dx == 1, iota, idx_p1))
                snbr[pl.ds(i * 16, 16)] = plsc.load_gather(spf3, [base + idxv])
                return _c
            lax.fori_loop(0, 72, gbody, 0)

            def nbody(k, _c):
                acc = zero
                for cc in range(8):
                    sv = snbr[pl.ds((k * 8 + cc) * 16, 16)]
                    acc = acc + sv * sv
                snbr[pl.ds((72 + k) * 16, 16)] = acc
                return _c
            lax.fori_loop(0, 9, nbody, 0)

            def p1(p, _c, b=b):
                fs = [ftile[b, pl.ds(p * 128 + cc * 16, 16)] for cc in range(8)]
                vals = []
                for k in range(9):
                    u = zero
                    for cc in range(8):
                        u = u + fs[cc] * snbr[pl.ds((k * 8 + cc) * 16, 16)]
                    vals.append(u + u - snbr[pl.ds((72 + k) * 16, 16)])
                m = vals[0]
                for k in range(1, 9):
                    m = jnp.maximum(m, vals[k])
                es = [jnp.exp(v - m) for v in vals]
                tot = es[0]
                for k in range(1, 9):
                    tot = tot + es[k]
                rinv = 1.0 / tot
                for k in range(9):
                    qtile[pl.ds(k * (_NP * 16) + p * 16, 16)] = es[k] * rinv
                return _c
            lax.fori_loop(0, _NP, p1, 0)

            if t == 4:
                for k in range(9):
                    def qlbody(p, _c, k=k):
                        v = qtile[pl.ds(k * 3136 + p * 16, 16)]
                        py = p // 14
                        px = p - py * 14
                        plsc.store_scatter(
                            imgbuf, [iota14 + (py * 224 + px)], v)
                        return _c
                    lax.fori_loop(0, _NP, qlbody, 0)
                    pltpu.sync_copy(imgbuf.at[pl.ds(0, 3200)],
                                    q_hbm.at[bg, k, s])

            for i in range(27):
                part[pl.ds(i * 16, 16)] = zero

            def p2c(kc, _c, b=b):
                def p2(p, accs):
                    fs = [ftile[b, pl.ds(p * 128 + cc * 16, 16)]
                          for cc in range(8)]
                    new = []
                    for j in range(3):
                        k = kc * 3 + j
                        q = qtile[pl.ds(k * (_NP * 16) + p * 16, 16)]
                        a = accs[j * 9:(j + 1) * 9]
                        new += [a[cc] + q * fs[cc] for cc in range(8)]
                        new.append(a[8] + q)
                    return tuple(new)
                accs = lax.fori_loop(0, _NP, p2, (zero,) * 27)
                for i in range(27):
                    kacc[pl.ds((kc * 27 + i) * 16, 16)] = accs[i]
                return _c
            lax.fori_loop(0, 3, p2c, 0)

            def p2k(k, _c):
                dy = k // 3
                dx = k - dy * 3
                for ch in range(9):
                    tmp[ch, :] = kacc[pl.ds((k * 9 + ch) * 16, 16)]
                for ch in range(9):
                    chf = jnp.full((16,), ch, jnp.int32)
                    v = tmp[ch, :]
                    gA = plsc.load_gather(tmp, [chf, idx_p1])
                    shA = (jnp.where(iota <= 14, gA, 0.0)
                           + jnp.where(iota == 0, v, 0.0))
                    gB = plsc.load_gather(tmp, [chf, idx_m1])
                    shB = (jnp.where(iota >= 1, gB, 0.0)
                           + jnp.where(iota == 15, v, 0.0))
                    sh = jnp.where(dx == 0, shA, jnp.where(dx == 2, shB, v))
                    off = (dy * 9 + ch) * 16
                    part[pl.ds(off, 16)] = part[pl.ds(off, 16)] + sh
                return _c
            lax.fori_loop(0, 9, p2k, 0)

            is0 = s == 0
            is15 = s == 15
            for ch in range(9):
                p0 = part[pl.ds(ch * 16, 16)]
                p1v = part[pl.ds((9 + ch) * 16, 16)]
                p2v = part[pl.ds((18 + ch) * 16, 16)]
                p1n = (p1v + jnp.where(is0, p0, zero)
                       + jnp.where(is15, p2v, zero))
                part[pl.ds(ch * 16, 16)] = jnp.where(is0, zero, p0)
                part[pl.ds((9 + ch) * 16, 16)] = p1n
                part[pl.ds((18 + ch) * 16, 16)] = jnp.where(is15, zero, p2v)
            pltpu.sync_copy(part, shp.at[pl.ds((s * 2 + b) * 432, 432)])
        plsc.subcore_barrier()

        for b in range(2):
            bg = 2 * c + b
            pltpu.sync_copy(shp.at[pl.ds((s * 2 + b) * 432 + 144, 144)],
                            rball.at[pl.ds(0, 144)])
            pltpu.sync_copy(shp.at[pl.ds((sm1 * 2 + b) * 432 + 288, 144)],
                            rball.at[pl.ds(144, 144)])
            pltpu.sync_copy(shp.at[pl.ds((sp1 * 2 + b) * 432, 144)],
                            rball.at[pl.ds(288, 144)])
            up_ok = s > 0
            dn_ok = s < 15

            def rd(ch):
                a0 = rball[pl.ds(ch * 16, 16)]
                a1 = jnp.where(up_ok, rball[pl.ds((9 + ch) * 16, 16)], zero)
                a2 = jnp.where(dn_ok, rball[pl.ds((18 + ch) * 16, 16)], zero)
                return a0 + a1 + a2
            rinv = 1.0 / (rd(8) + 1e-8)
            for cc in range(8):
                spfrow[pl.ds((b * 8 + cc) * 16, 16)] = rd(cc) * rinv
            pltpu.sync_copy(spfrow.at[pl.ds(b * 128, 128)],
                            sspf.at[pl.ds((b * 16 + s) * 128, 128)])
            if t == 4:
                pltpu.sync_copy(spfrow.at[pl.ds(b * 128, 128)], spf_hbm.at[bg, s])
        plsc.subcore_barrier()


def _cluster_sc(fcell, interpret=False):
    mesh = plsc.VectorSubcoreMesh(core_axis_name="c", subcore_axis_name="s")
    f = pl.kernel(
        _cluster_body,
        out_type=[jax.ShapeDtypeStruct((4, 9, 16, 3200), jnp.float32),
                  jax.ShapeDtypeStruct((4, 16, 128), jnp.float32)],
        mesh=mesh,
        scratch_types=[
            pltpu.VMEM((2, _NP * 128), jnp.float32),
            pltpu.VMEM((25600,), jnp.float32),
            pltpu.VMEM((9 * _NP * 16,), jnp.float32),
            pltpu.VMEM((81 * 16,), jnp.float32),
            pltpu.VMEM((24 * 16,), jnp.float32),
            pltpu.VMEM((27 * 16,), jnp.float32),
            pltpu.VMEM((9, 16), jnp.float32),
            pltpu.VMEM((81 * 16,), jnp.float32),
            pltpu.VMEM((27 * 16,), jnp.float32),
            pltpu.VMEM((16 * 16,), jnp.float32),
            pltpu.VMEM_SHARED((32 * 27 * 16,), jnp.float32),
            pltpu.VMEM_SHARED((2 * 16 * 128,), jnp.float32),
        ],
        compiler_params=pltpu.CompilerParams(needs_layout_passes=False),
        interpret=interpret,
    )
    return f(fcell)


def kernel(featlist, Wconv, bconv):
    x = featlist[0]
    featall = _conv_pallas(x, Wconv, bconv)
    B, C, H, W = featall.shape
    f_pad = jnp.pad(featall.reshape(B, C, 16, 3136),
                    ((0, 0), (0, 0), (0, 0), (0, 64)))
    q_img, spf_cells = _cluster_sc(f_pad)
    q_img = q_img[:, :, :, :3136]
    Qout = q_img.reshape(B, 9, H, W)
    spf_out = (spf_cells.reshape(B, 16, 8, 16)
               .transpose(0, 1, 3, 2).reshape(B, 256, C))
    return (Qout, spf_out, featall)
